# baseline (speedup 1.0000x reference)
import functools

import jax
import jax.numpy as jnp
import numpy as np
from jax import lax
from jax.experimental import pallas as pl
from jax.experimental.pallas import tpu as pltpu
from jax.experimental.pallas import tpu_sc as plsc

VOCAB = 100000
DIM = 64
B = 16384
NUM_NEG = 100

VPAD = 100352
VROWS = VPAD // 128

_NC = 2
_NS = 16
_NW = _NC * _NS
_BPW = B // _NW

_TINY = np.float32(np.finfo(np.float32).tiny)


def _threefry_bits(x1):
    ks0 = jnp.uint32(0)
    ks1 = jnp.uint32(42)
    ks2 = jnp.uint32(0x1BD11BDA) ^ ks0 ^ ks1
    rot = ((13, 15, 26, 6), (17, 29, 16, 24))
    ks = (ks0, ks1, ks2)
    x0 = jnp.zeros_like(x1) + ks0
    x1 = x1 + ks1
    for blk in range(5):
        for r in rot[blk % 2]:
            x0 = x0 + x1
            x1 = ((x1 << r) | (x1 >> (32 - r))) ^ x0
        x0 = x0 + ks[(blk + 1) % 3]
        x1 = x1 + ks[(blk + 2) % 3] + jnp.uint32(blk + 1)
    return x0 ^ x1


def _expected(p):
    t = jnp.float32(NUM_NEG) * (-p * (1.0 + p * (0.5 + p * (1.0 / 3.0))))
    return -t * (1.0 + t * (0.5 + t * (1.0 / 6.0)))


def _vocab_body(cext_ref, bias_ref, g_ref, adj_ref):
    c = cext_ref[...]
    rid = lax.broadcasted_iota(jnp.int32, c.shape, 0)
    cid = lax.broadcasted_iota(jnp.int32, c.shape, 1)
    gid = rid * 128 + cid
    valid = (gid >= 1) & (gid <= VOCAB)
    logpz = jnp.float32(0.4) * jnp.log(c)
    pz = jnp.where(valid, jnp.exp(logpz), jnp.float32(0.0))
    s = jnp.sum(pz)
    adj = bias_ref[...] - jnp.log(_expected(pz / s))
    adj_ref[...] = jnp.where(valid, adj, jnp.float32(0.0))
    bits = _threefry_bits(gid.astype(jnp.uint32))
    fb = lax.bitcast_convert_type(
        (bits >> 9) | jnp.uint32(0x3F800000), jnp.float32) - jnp.float32(1.0)
    u = jnp.maximum(fb + _TINY, _TINY)
    gum = -jnp.log(-jnp.log(u))
    g_ref[...] = jnp.where(valid, gum + logpz, jnp.float32(-3e38))


def _vocab_call(cext, bpad):
    return pl.pallas_call(
        _vocab_body,
        in_specs=[
            pl.BlockSpec((VROWS, 128), lambda: (0, 0)),
            pl.BlockSpec((VROWS, 128), lambda: (0, 0)),
        ],
        out_specs=[
            pl.BlockSpec((VROWS, 128), lambda: (0, 0)),
            pl.BlockSpec((VROWS, 128), lambda: (0, 0)),
        ],
        out_shape=(
            jax.ShapeDtypeStruct((VROWS, 128), jnp.float32),
            jax.ShapeDtypeStruct((VROWS, 128), jnp.float32),
        ),
    )(cext, bpad)


def _sc_gather_body(table_hbm, adj_hbm, idx_hbm, samp_hbm, big_out, sp_out,
                    idx_v, rows_v, adj_v, big_v, sidx_v, srows_v, sadj_v, sbig_v,
                    sem_a, sem_b, sem_c, sem_d):
    wid = lax.axis_index("s") * _NC + lax.axis_index("c")
    base = wid * _BPW
    pltpu.sync_copy(idx_hbm.at[pl.ds(base, _BPW)], idx_v)
    cp_a = pltpu.async_copy(table_hbm.at[idx_v], rows_v, sem_a)
    cp_b = pltpu.async_copy(adj_hbm.at[idx_v], adj_v, sem_b)
    cp_a.wait()
    cp_b.wait()

    def move16(j, carry):
        vals = adj_v[pl.ds(j * 16, 16)]
        for k in range(16):
            r = j * 16 + k
            big_v[r, pl.ds(0, 16)] = rows_v[r, pl.ds(0, 16)]
            big_v[r, pl.ds(16, 16)] = rows_v[r, pl.ds(16, 16)]
            big_v[r, pl.ds(32, 16)] = rows_v[r, pl.ds(32, 16)]
            big_v[r, pl.ds(48, 16)] = rows_v[r, pl.ds(48, 16)]
            big_v[r, pl.ds(64, 16)] = jnp.full((16,), vals[k], jnp.float32)
        return carry
    lax.fori_loop(0, _BPW // 16, move16, 0)
    pltpu.sync_copy(big_v, big_out.at[pl.ds(base, _BPW)])

    @pl.when(wid == 0)
    def _():
        pltpu.sync_copy(samp_hbm, sidx_v)
        cp_c = pltpu.async_copy(table_hbm.at[sidx_v], srows_v, sem_c)
        cp_d = pltpu.async_copy(adj_hbm.at[sidx_v], sadj_v, sem_d)
        cp_c.wait()
        cp_d.wait()

        def smove16(j, carry):
            svals = sadj_v[pl.ds(j * 16, 16)]
            for k in range(16):
                r = j * 16 + k
                sbig_v[r, pl.ds(0, 16)] = srows_v[r, pl.ds(0, 16)]
                sbig_v[r, pl.ds(16, 16)] = srows_v[r, pl.ds(16, 16)]
                sbig_v[r, pl.ds(32, 16)] = srows_v[r, pl.ds(32, 16)]
                sbig_v[r, pl.ds(48, 16)] = srows_v[r, pl.ds(48, 16)]
                sbig_v[r, pl.ds(64, 16)] = jnp.full((16,), svals[k], jnp.float32)
            return carry
        lax.fori_loop(0, 8, smove16, 0)
        pltpu.sync_copy(sbig_v, sp_out)


@functools.lru_cache(maxsize=1)
def _sc_gather_kernel():
    return pl.kernel(
        _sc_gather_body,
        mesh=plsc.VectorSubcoreMesh(core_axis_name="c", subcore_axis_name="s"),
        compiler_params=pltpu.CompilerParams(use_tc_tiling_on_sc=False),
        out_type=(
            jax.ShapeDtypeStruct((B, 128), jnp.float32),
            jax.ShapeDtypeStruct((128, 128), jnp.float32),
        ),
        scratch_types=[
            pltpu.VMEM((_BPW,), jnp.int32),
            pltpu.VMEM((_BPW, DIM), jnp.float32),
            pltpu.VMEM((_BPW,), jnp.float32),
            pltpu.VMEM((_BPW, 128), jnp.float32),
            pltpu.VMEM((128,), jnp.int32),
            pltpu.VMEM((128, DIM), jnp.float32),
            pltpu.VMEM((128,), jnp.float32),
            pltpu.VMEM((128, 128), jnp.float32),
            pltpu.SemaphoreType.DMA,
            pltpu.SemaphoreType.DMA,
            pltpu.SemaphoreType.DMA,
            pltpu.SemaphoreType.DMA,
        ],
    )


_BB = 1024
_GRID = B // _BB


def _batch_body(inp_ref, big_ref, sp_ref, out_ref, loss_ref):
    i = pl.program_id(0)
    x = inp_ref[...]
    xa = jnp.concatenate(
        [x, jnp.ones((x.shape[0], 1), jnp.float32)], axis=1)
    twa = big_ref[...][:, :DIM + 1]
    tl = jnp.sum(xa * twa, axis=1, keepdims=True)

    spa = sp_ref[...][:, :DIM + 1]
    sl = lax.dot_general(xa, spa, (((1,), (1,)), ((), ())),
                         preferred_element_type=jnp.float32)

    z = sl - tl
    sp = jnp.maximum(z, 0.0) + jnp.log(1.0 + jnp.exp(-jnp.abs(z)))
    colmask = lax.broadcasted_iota(jnp.int32, sp.shape, 1) < NUM_NEG
    part = jnp.sum(jnp.where(colmask, sp, 0.0)) * jnp.float32(1.0 / (B * NUM_NEG))

    @pl.when(i == 0)
    def _():
        loss_ref[0, 0] = jnp.float32(0.0)

    loss_ref[0, 0] += part
    out_ref[...] = jnp.concatenate([tl, sl[:, :127]], axis=1)


def _batch_call(inp, big, sp):
    return pl.pallas_call(
        _batch_body,
        grid=(_GRID,),
        in_specs=[
            pl.BlockSpec((_BB, DIM), lambda i: (i, 0)),
            pl.BlockSpec((_BB, 128), lambda i: (i, 0)),
            pl.BlockSpec((128, 128), lambda i: (0, 0)),
        ],
        out_specs=[
            pl.BlockSpec((_BB, 128), lambda i: (i, 0)),
            pl.BlockSpec(memory_space=pltpu.SMEM),
        ],
        out_shape=(
            jax.ShapeDtypeStruct((B, 128), jnp.float32),
            jax.ShapeDtypeStruct((1, 1), jnp.float32),
        ),
    )(inp, big, sp)


def kernel(label, inputs, table, biases, counts):
    cext = jnp.concatenate(
        [jnp.ones((1,), jnp.float32), counts,
         jnp.ones((VPAD - VOCAB - 1,), jnp.float32)]).reshape(VROWS, 128)
    bpad = jnp.pad(biases, (0, VPAD - VOCAB - 1)).reshape(VROWS, 128)
    g2d, adj2d = _vocab_call(cext, bpad)

    g_flat = g2d.reshape(-1)
    _, samp = lax.top_k(g_flat, NUM_NEG)
    samp = samp.astype(jnp.int32)
    samp_pad = jnp.concatenate([samp, jnp.zeros((28,), jnp.int32)])

    adj_flat = adj2d.reshape(-1)
    big, sp = _sc_gather_kernel()(table, adj_flat, label, samp_pad)

    logits_pad, loss = _batch_call(inputs, big, sp)
    return logits_pad[:, :NUM_NEG + 1], loss[0, 0]


# speedup vs baseline: 1.8704x; 1.8704x over previous
import functools

import jax
import jax.numpy as jnp
import numpy as np
from jax import lax
from jax.experimental import pallas as pl
from jax.experimental.pallas import tpu as pltpu
from jax.experimental.pallas import tpu_sc as plsc

VOCAB = 100000
DIM = 64
B = 16384
NUM_NEG = 100

VPAD = 100352
VROWS = VPAD // 128

_NC = 2
_NS = 16
_NW = _NC * _NS
_BPW = B // _NW

_TINY = np.float32(np.finfo(np.float32).tiny)


def _threefry_bits(x1):
    ks0 = jnp.uint32(0)
    ks1 = jnp.uint32(42)
    ks2 = jnp.uint32(0x1BD11BDA) ^ ks0 ^ ks1
    rot = ((13, 15, 26, 6), (17, 29, 16, 24))
    ks = (ks0, ks1, ks2)
    x0 = jnp.zeros_like(x1) + ks0
    x1 = x1 + ks1
    for blk in range(5):
        for r in rot[blk % 2]:
            x0 = x0 + x1
            x1 = ((x1 << r) | (x1 >> (32 - r))) ^ x0
        x0 = x0 + ks[(blk + 1) % 3]
        x1 = x1 + ks[(blk + 2) % 3] + jnp.uint32(blk + 1)
    return x0 ^ x1


def _expected(p):
    t = jnp.float32(NUM_NEG) * (-p * (1.0 + p * (0.5 + p * (1.0 / 3.0))))
    return -t * (1.0 + t * (0.5 + t * (1.0 / 6.0)))


def _vocab_body(cext_ref, bias_ref, g_ref, adj_ref):
    c = cext_ref[...]
    rid = lax.broadcasted_iota(jnp.int32, c.shape, 0)
    cid = lax.broadcasted_iota(jnp.int32, c.shape, 1)
    gid = rid * 128 + cid
    valid = (gid >= 1) & (gid <= VOCAB)
    logpz = jnp.float32(0.4) * jnp.log(c)
    pz = jnp.where(valid, jnp.exp(logpz), jnp.float32(0.0))
    s = jnp.sum(pz)
    adj = bias_ref[...] - jnp.log(_expected(pz / s))
    adj_ref[...] = jnp.where(valid, adj, jnp.float32(0.0))
    bits = _threefry_bits(gid.astype(jnp.uint32))
    fb = lax.bitcast_convert_type(
        (bits >> 9) | jnp.uint32(0x3F800000), jnp.float32) - jnp.float32(1.0)
    u = jnp.maximum(fb + _TINY, _TINY)
    gum = -jnp.log(-jnp.log(u))
    g_ref[...] = jnp.where(valid, gum + logpz, jnp.float32(-3e38))


def _vocab_call(cext, bpad):
    return pl.pallas_call(
        _vocab_body,
        in_specs=[
            pl.BlockSpec((VROWS, 128), lambda: (0, 0)),
            pl.BlockSpec((VROWS, 128), lambda: (0, 0)),
        ],
        out_specs=[
            pl.BlockSpec((VROWS, 128), lambda: (0, 0)),
            pl.BlockSpec((VROWS, 128), lambda: (0, 0)),
        ],
        out_shape=(
            jax.ShapeDtypeStruct((VROWS, 128), jnp.float32),
            jax.ShapeDtypeStruct((VROWS, 128), jnp.float32),
        ),
    )(cext, bpad)


def _sc_gather_body(table_hbm, adj_hbm, idx_hbm, samp_hbm, big_out, sp_out,
                    idx_v, rows_v, adj_v, big_v, sidx_v, srows_v, sadj_v, sbig_v,
                    sem_a, sem_b, sem_c, sem_d):
    wid = lax.axis_index("s") * _NC + lax.axis_index("c")
    base = wid * _BPW
    pltpu.sync_copy(idx_hbm.at[pl.ds(base, _BPW)], idx_v)
    cp_a = pltpu.async_copy(table_hbm.at[idx_v], rows_v, sem_a)
    cp_b = pltpu.async_copy(adj_hbm.at[idx_v], adj_v, sem_b)
    cp_a.wait()
    cp_b.wait()

    def move16(j, carry):
        vals = adj_v[pl.ds(j * 16, 16)]
        for k in range(16):
            r = j * 16 + k
            big_v[r, pl.ds(0, 16)] = rows_v[r, pl.ds(0, 16)]
            big_v[r, pl.ds(16, 16)] = rows_v[r, pl.ds(16, 16)]
            big_v[r, pl.ds(32, 16)] = rows_v[r, pl.ds(32, 16)]
            big_v[r, pl.ds(48, 16)] = rows_v[r, pl.ds(48, 16)]
            big_v[r, pl.ds(64, 16)] = jnp.full((16,), vals[k], jnp.float32)
        return carry
    lax.fori_loop(0, _BPW // 16, move16, 0)
    pltpu.sync_copy(big_v, big_out.at[pl.ds(base, _BPW)])

    @pl.when(wid == 0)
    def _():
        pltpu.sync_copy(samp_hbm, sidx_v)
        cp_c = pltpu.async_copy(table_hbm.at[sidx_v], srows_v, sem_c)
        cp_d = pltpu.async_copy(adj_hbm.at[sidx_v], sadj_v, sem_d)
        cp_c.wait()
        cp_d.wait()

        def smove16(j, carry):
            svals = sadj_v[pl.ds(j * 16, 16)]
            for k in range(16):
                r = j * 16 + k
                sbig_v[r, pl.ds(0, 16)] = srows_v[r, pl.ds(0, 16)]
                sbig_v[r, pl.ds(16, 16)] = srows_v[r, pl.ds(16, 16)]
                sbig_v[r, pl.ds(32, 16)] = srows_v[r, pl.ds(32, 16)]
                sbig_v[r, pl.ds(48, 16)] = srows_v[r, pl.ds(48, 16)]
                sbig_v[r, pl.ds(64, 16)] = jnp.full((16,), svals[k], jnp.float32)
            return carry
        lax.fori_loop(0, 8, smove16, 0)
        pltpu.sync_copy(sbig_v, sp_out)


@functools.lru_cache(maxsize=1)
def _sc_gather_kernel():
    return pl.kernel(
        _sc_gather_body,
        mesh=plsc.VectorSubcoreMesh(core_axis_name="c", subcore_axis_name="s"),
        compiler_params=pltpu.CompilerParams(use_tc_tiling_on_sc=False),
        out_type=(
            jax.ShapeDtypeStruct((B, 128), jnp.float32),
            jax.ShapeDtypeStruct((128, 128), jnp.float32),
        ),
        scratch_types=[
            pltpu.VMEM((_BPW,), jnp.int32),
            pltpu.VMEM((_BPW, DIM), jnp.float32),
            pltpu.VMEM((_BPW,), jnp.float32),
            pltpu.VMEM((_BPW, 128), jnp.float32),
            pltpu.VMEM((128,), jnp.int32),
            pltpu.VMEM((128, DIM), jnp.float32),
            pltpu.VMEM((128,), jnp.float32),
            pltpu.VMEM((128, 128), jnp.float32),
            pltpu.SemaphoreType.DMA,
            pltpu.SemaphoreType.DMA,
            pltpu.SemaphoreType.DMA,
            pltpu.SemaphoreType.DMA,
        ],
    )


_SNS = 16
_SCHUNK = VPAD // _SNS
_SVREGS = _SCHUNK // 16


def _mono16(gvec):
    mi = lax.bitcast_convert_type(gvec, jnp.int32)
    neg = lax.bitcast_convert_type(mi >> 31, jnp.uint32)
    return lax.bitcast_convert_type(mi, jnp.uint32) ^ (neg | jnp.uint32(0x80000000))


def _valat(vec, idx_splat):
    i16 = lax.iota(jnp.int32, 16)
    return jnp.sum(jnp.where(i16 == idx_splat, vec, 0))


def _sc_select_body(g_hbm, samp_out,
                    gv, hist_v, tot_v, gall_v, gtot_v, res_v,
                    selu_v, seli_v, tselu_v, tseli_v, rankb_v, trank_v, samp_v,
                    sh_tot, sh_res, sh_selu, sh_seli, sh_rank,
                    sem_a, sem_b):
    wid = lax.axis_index("s")
    base = wid * _SCHUNK
    pltpu.sync_copy(g_hbm.at[pl.ds(base, _SCHUNK)], gv)
    i16 = lax.iota(jnp.int32, 16)
    ones16 = jnp.ones((16,), jnp.int32)
    zeros16 = jnp.zeros((16,), jnp.int32)

    prefix = jnp.zeros((16,), jnp.uint32)
    kk = jnp.full((16,), NUM_NEG, jnp.int32)

    for rnd in range(2):
        sh = 24 - 8 * rnd

        def zro(j, c):
            hist_v[pl.ds(j * 16, 16)] = zeros16
            return c
        lax.fori_loop(0, 256, zro, 0)

        if rnd == 0:
            def scan(j, c):
                u = _mono16(gv[pl.ds(j * 16, 16)])
                b = ((u >> jnp.uint32(sh)) & jnp.uint32(0xFF)).astype(jnp.int32)
                plsc.addupdate_scatter(hist_v, [i16 * 256 + b], ones16)
                return c
        else:
            pref_hi = prefix >> jnp.uint32(sh + 8)

            def scan(j, c):
                u = _mono16(gv[pl.ds(j * 16, 16)])
                b = ((u >> jnp.uint32(sh)) & jnp.uint32(0xFF)).astype(jnp.int32)
                m = (u >> jnp.uint32(sh + 8)) == pref_hi
                plsc.addupdate_scatter(hist_v, [i16 * 256 + b], ones16, mask=m)
                return c
        lax.fori_loop(0, _SVREGS, scan, 0)

        def lred(b, c):
            acc = zeros16
            for l in range(16):
                acc = acc + hist_v[pl.ds(l * 256 + b * 16, 16)]
            tot_v[pl.ds(b * 16, 16)] = acc
            return c
        lax.fori_loop(0, 16, lred, 0)
        pltpu.sync_copy(tot_v, sh_tot.at[wid])
        plsc.subcore_barrier()

        @pl.when(wid == 0)
        def _():
            pltpu.sync_copy(sh_tot, gall_v)

            def gred(b, c):
                acc = zeros16
                for t in range(16):
                    acc = acc + gall_v[t, pl.ds(b * 16, 16)]
                gtot_v[pl.ds(b * 16, 16)] = acc
                return c
            lax.fori_loop(0, 16, gred, 0)

            gs = zeros16
            for j in range(16):
                gs = gs + plsc.load_gather(gtot_v, [i16 * 16 + j])
            cumgs = plsc.cumsum(gs)
            tot_all = jnp.full((16,), cumgs[15], jnp.int32)
            sufg = tot_all - cumgs + gs
            gstar = plsc.all_reduce_population_count(sufg >= kk) - 1
            above_g = tot_all - jnp.full((16,), _valat(cumgs, gstar), jnp.int32)
            sub = gtot_v[pl.ds(gstar[0] * 16, 16)]
            cumsub = plsc.cumsum(sub)
            sub_tot = jnp.full((16,), cumsub[15], jnp.int32)
            sbin = sub_tot - cumsub + sub + above_g
            jstar = plsc.all_reduce_population_count(sbin >= kk) - 1
            byte = gstar * 16 + jstar
            cnt_above = (jnp.full((16,), _valat(sbin, jstar), jnp.int32)
                         - jnp.full((16,), _valat(sub, jstar), jnp.int32))
            k_new = kk - cnt_above
            pref_new = prefix | (byte.astype(jnp.uint32) << jnp.uint32(sh))
            res_v[pl.ds(0, 16)] = lax.bitcast_convert_type(pref_new, jnp.int32)
            res_v[pl.ds(16, 16)] = k_new
            pltpu.sync_copy(res_v, sh_res)

        plsc.subcore_barrier()
        pltpu.sync_copy(sh_res, res_v)
        prefix = lax.bitcast_convert_type(res_v[pl.ds(0, 16)], jnp.uint32)
        kk = res_v[pl.ds(16, 16)]

    p16 = prefix >> jnp.uint32(16)
    selu_v[pl.ds(0, 16)] = jnp.zeros((16,), jnp.uint32)
    selu_v[pl.ds(16, 16)] = jnp.zeros((16,), jnp.uint32)
    seli_v[pl.ds(0, 16)] = zeros16
    seli_v[pl.ds(16, 16)] = zeros16

    def ext(j, off):
        u = _mono16(gv[pl.ds(j * 16, 16)])
        m = (u >> jnp.uint32(16)) >= p16
        plsc.store_compressed(selu_v.at[pl.ds(off, 16)], u, mask=m)
        gidx = jnp.full((16,), base, jnp.int32) + j * 16 + i16
        plsc.store_compressed(seli_v.at[pl.ds(off, 16)], gidx, mask=m)
        cnt = plsc.all_reduce_population_count(m)
        return jnp.minimum(off + cnt[0], 16)
    lax.fori_loop(0, _SVREGS, ext, jnp.int32(0))

    pltpu.sync_copy(selu_v, sh_selu.at[wid])
    pltpu.sync_copy(seli_v, sh_seli.at[wid])
    plsc.subcore_barrier()
    pltpu.sync_copy(sh_selu, tselu_v)
    pltpu.sync_copy(sh_seli, tseli_v)

    def z128(j, c):
        rankb_v[pl.ds(j * 16, 16)] = zeros16
        return c
    lax.fori_loop(0, 8, z128, 0)

    myu = selu_v[pl.ds(0, 16)]
    myidx = seli_v[pl.ds(0, 16)]
    ou = [tselu_v[t, pl.ds(v * 16, 16)] for t in range(16) for v in range(2)]
    oi = [tseli_v[t, pl.ds(v * 16, 16)] for t in range(16) for v in range(2)]
    ranks = zeros16
    for lane in range(16):
        us = jnp.full((16,), myu[lane], jnp.uint32)
        ix = jnp.full((16,), myidx[lane], jnp.int32)
        acc = zeros16
        for q in range(32):
            gt = ou[q] > us
            tie = (ou[q] == us) & (oi[q] < ix)
            acc = acc + gt.astype(jnp.int32) + tie.astype(jnp.int32)
        ranks = jnp.where(i16 == lane, jnp.full((16,), jnp.sum(acc), jnp.int32),
                          ranks)
    plsc.store_scatter(rankb_v, [ranks], myidx, mask=myu != jnp.uint32(0))
    pltpu.sync_copy(rankb_v, sh_rank.at[wid])
    plsc.subcore_barrier()

    @pl.when(wid == 0)
    def _():
        pltpu.sync_copy(sh_rank, trank_v)
        for v in range(8):
            acc = zeros16
            for t in range(16):
                acc = acc + trank_v[t, pl.ds(v * 16, 16)]
            samp_v[pl.ds(v * 16, 16)] = acc
        pltpu.sync_copy(samp_v, samp_out)


@functools.lru_cache(maxsize=1)
def _sc_select_kernel():
    return pl.kernel(
        _sc_select_body,
        mesh=plsc.VectorSubcoreMesh(core_axis_name="c", subcore_axis_name="s",
                                    num_cores=1),
        compiler_params=pltpu.CompilerParams(use_tc_tiling_on_sc=False,
                                             needs_layout_passes=False),
        out_type=jax.ShapeDtypeStruct((128,), jnp.int32),
        scratch_types=[
            pltpu.VMEM((_SCHUNK,), jnp.float32),
            pltpu.VMEM((4096,), jnp.int32),
            pltpu.VMEM((256,), jnp.int32),
            pltpu.VMEM((16, 256), jnp.int32),
            pltpu.VMEM((256,), jnp.int32),
            pltpu.VMEM((32,), jnp.int32),
            pltpu.VMEM((32,), jnp.uint32),
            pltpu.VMEM((32,), jnp.int32),
            pltpu.VMEM((16, 32), jnp.uint32),
            pltpu.VMEM((16, 32), jnp.int32),
            pltpu.VMEM((128,), jnp.int32),
            pltpu.VMEM((16, 128), jnp.int32),
            pltpu.VMEM((128,), jnp.int32),
            pltpu.VMEM_SHARED((16, 256), jnp.int32),
            pltpu.VMEM_SHARED((32,), jnp.int32),
            pltpu.VMEM_SHARED((16, 32), jnp.uint32),
            pltpu.VMEM_SHARED((16, 32), jnp.int32),
            pltpu.VMEM_SHARED((16, 128), jnp.int32),
            pltpu.SemaphoreType.DMA,
            pltpu.SemaphoreType.DMA,
        ],
    )


_BB = 1024
_GRID = B // _BB


def _batch_body(inp_ref, big_ref, sp_ref, out_ref, loss_ref):
    i = pl.program_id(0)
    x = inp_ref[...]
    xa = jnp.concatenate(
        [x, jnp.ones((x.shape[0], 1), jnp.float32)], axis=1)
    twa = big_ref[...][:, :DIM + 1]
    tl = jnp.sum(xa * twa, axis=1, keepdims=True)

    spa = sp_ref[...][:, :DIM + 1]
    sl = lax.dot_general(xa, spa, (((1,), (1,)), ((), ())),
                         preferred_element_type=jnp.float32)

    z = sl - tl
    sp = jnp.maximum(z, 0.0) + jnp.log(1.0 + jnp.exp(-jnp.abs(z)))
    colmask = lax.broadcasted_iota(jnp.int32, sp.shape, 1) < NUM_NEG
    part = jnp.sum(jnp.where(colmask, sp, 0.0)) * jnp.float32(1.0 / (B * NUM_NEG))

    @pl.when(i == 0)
    def _():
        loss_ref[0, 0] = jnp.float32(0.0)

    loss_ref[0, 0] += part
    out_ref[...] = jnp.concatenate([tl, sl[:, :127]], axis=1)


def _batch_call(inp, big, sp):
    return pl.pallas_call(
        _batch_body,
        grid=(_GRID,),
        in_specs=[
            pl.BlockSpec((_BB, DIM), lambda i: (i, 0)),
            pl.BlockSpec((_BB, 128), lambda i: (i, 0)),
            pl.BlockSpec((128, 128), lambda i: (0, 0)),
        ],
        out_specs=[
            pl.BlockSpec((_BB, 128), lambda i: (i, 0)),
            pl.BlockSpec(memory_space=pltpu.SMEM),
        ],
        out_shape=(
            jax.ShapeDtypeStruct((B, 128), jnp.float32),
            jax.ShapeDtypeStruct((1, 1), jnp.float32),
        ),
    )(inp, big, sp)


def kernel(label, inputs, table, biases, counts):
    cext = jnp.concatenate(
        [jnp.ones((1,), jnp.float32), counts,
         jnp.ones((VPAD - VOCAB - 1,), jnp.float32)]).reshape(VROWS, 128)
    bpad = jnp.pad(biases, (0, VPAD - VOCAB - 1)).reshape(VROWS, 128)
    g2d, adj2d = _vocab_call(cext, bpad)

    g_flat = g2d.reshape(-1)
    samp_pad = _sc_select_kernel()(g_flat)

    adj_flat = adj2d.reshape(-1)
    big, sp = _sc_gather_kernel()(table, adj_flat, label, samp_pad)

    logits_pad, loss = _batch_call(inputs, big, sp)
    return logits_pad[:, :NUM_NEG + 1], loss[0, 0]


# speedup vs baseline: 1.9371x; 1.0356x over previous
import functools

import jax
import jax.numpy as jnp
import numpy as np
from jax import lax
from jax.experimental import pallas as pl
from jax.experimental.pallas import tpu as pltpu
from jax.experimental.pallas import tpu_sc as plsc

VOCAB = 100000
DIM = 64
B = 16384
NUM_NEG = 100

VPAD = 100352
VROWS = VPAD // 128

_NC = 2
_NS = 16
_NW = _NC * _NS
_BPW = B // _NW

_TINY = np.float32(np.finfo(np.float32).tiny)


def _threefry_bits(x1):
    ks0 = jnp.uint32(0)
    ks1 = jnp.uint32(42)
    ks2 = jnp.uint32(0x1BD11BDA) ^ ks0 ^ ks1
    rot = ((13, 15, 26, 6), (17, 29, 16, 24))
    ks = (ks0, ks1, ks2)
    x0 = jnp.zeros_like(x1) + ks0
    x1 = x1 + ks1
    for blk in range(5):
        for r in rot[blk % 2]:
            x0 = x0 + x1
            x1 = ((x1 << r) | (x1 >> (32 - r))) ^ x0
        x0 = x0 + ks[(blk + 1) % 3]
        x1 = x1 + ks[(blk + 2) % 3] + jnp.uint32(blk + 1)
    return x0 ^ x1


def _expected(p):
    t = jnp.float32(NUM_NEG) * (-p * (1.0 + p * (0.5 + p * (1.0 / 3.0))))
    return -t * (1.0 + t * (0.5 + t * (1.0 / 6.0)))


def _vocab_body(cext_ref, bias_ref, g_ref, adj_ref):
    c = cext_ref[...]
    rid = lax.broadcasted_iota(jnp.int32, c.shape, 0)
    cid = lax.broadcasted_iota(jnp.int32, c.shape, 1)
    gid = rid * 128 + cid
    valid = (gid >= 1) & (gid <= VOCAB)
    logpz = jnp.float32(0.4) * jnp.log(c)
    pz = jnp.where(valid, jnp.exp(logpz), jnp.float32(0.0))
    s = jnp.sum(pz)
    adj = bias_ref[...] - jnp.log(_expected(pz / s))
    adj_ref[...] = jnp.where(valid, adj, jnp.float32(0.0))
    bits = _threefry_bits(gid.astype(jnp.uint32))
    fb = lax.bitcast_convert_type(
        (bits >> 9) | jnp.uint32(0x3F800000), jnp.float32) - jnp.float32(1.0)
    u = jnp.maximum(fb + _TINY, _TINY)
    gum = -jnp.log(-jnp.log(u))
    g_ref[...] = jnp.where(valid, gum + logpz, jnp.float32(-3e38))


def _vocab_call(cext, bpad):
    return pl.pallas_call(
        _vocab_body,
        in_specs=[
            pl.BlockSpec((VROWS, 128), lambda: (0, 0)),
            pl.BlockSpec((VROWS, 128), lambda: (0, 0)),
        ],
        out_specs=[
            pl.BlockSpec((VROWS, 128), lambda: (0, 0)),
            pl.BlockSpec((VROWS, 128), lambda: (0, 0)),
        ],
        out_shape=(
            jax.ShapeDtypeStruct((VROWS, 128), jnp.float32),
            jax.ShapeDtypeStruct((VROWS, 128), jnp.float32),
        ),
    )(cext, bpad)


def _sc_gather_body(table_hbm, adj_hbm, idx_hbm, samp_hbm, big_out, sp_out,
                    idx_v, rows_v, adj_v, big_v, sidx_v, srows_v, sadj_v, sbig_v,
                    sem_a, sem_b, sem_c, sem_d):
    wid = lax.axis_index("s") * _NC + lax.axis_index("c")
    base = wid * _BPW
    pltpu.sync_copy(idx_hbm.at[pl.ds(base, _BPW)], idx_v)
    cp_a = pltpu.async_copy(table_hbm.at[idx_v], rows_v, sem_a)
    cp_b = pltpu.async_copy(adj_hbm.at[idx_v], adj_v, sem_b)
    cp_a.wait()
    cp_b.wait()

    def move16(j, carry):
        vals = adj_v[pl.ds(j * 16, 16)]
        for k in range(16):
            r = j * 16 + k
            big_v[r, pl.ds(0, 16)] = rows_v[r, pl.ds(0, 16)]
            big_v[r, pl.ds(16, 16)] = rows_v[r, pl.ds(16, 16)]
            big_v[r, pl.ds(32, 16)] = rows_v[r, pl.ds(32, 16)]
            big_v[r, pl.ds(48, 16)] = rows_v[r, pl.ds(48, 16)]
            big_v[r, pl.ds(64, 16)] = jnp.full((16,), vals[k], jnp.float32)
        return carry
    lax.fori_loop(0, _BPW // 16, move16, 0)
    pltpu.sync_copy(big_v, big_out.at[pl.ds(base, _BPW)])

    @pl.when(wid == 0)
    def _():
        pltpu.sync_copy(samp_hbm, sidx_v)
        cp_c = pltpu.async_copy(table_hbm.at[sidx_v], srows_v, sem_c)
        cp_d = pltpu.async_copy(adj_hbm.at[sidx_v], sadj_v, sem_d)
        cp_c.wait()
        cp_d.wait()

        def smove16(j, carry):
            svals = sadj_v[pl.ds(j * 16, 16)]
            for k in range(16):
                r = j * 16 + k
                sbig_v[r, pl.ds(0, 16)] = srows_v[r, pl.ds(0, 16)]
                sbig_v[r, pl.ds(16, 16)] = srows_v[r, pl.ds(16, 16)]
                sbig_v[r, pl.ds(32, 16)] = srows_v[r, pl.ds(32, 16)]
                sbig_v[r, pl.ds(48, 16)] = srows_v[r, pl.ds(48, 16)]
                sbig_v[r, pl.ds(64, 16)] = jnp.full((16,), svals[k], jnp.float32)
            return carry
        lax.fori_loop(0, 8, smove16, 0)
        pltpu.sync_copy(sbig_v, sp_out)


@functools.lru_cache(maxsize=1)
def _sc_gather_kernel():
    return pl.kernel(
        _sc_gather_body,
        mesh=plsc.VectorSubcoreMesh(core_axis_name="c", subcore_axis_name="s"),
        compiler_params=pltpu.CompilerParams(use_tc_tiling_on_sc=False),
        out_type=(
            jax.ShapeDtypeStruct((B, 128), jnp.float32),
            jax.ShapeDtypeStruct((128, 128), jnp.float32),
        ),
        scratch_types=[
            pltpu.VMEM((_BPW,), jnp.int32),
            pltpu.VMEM((_BPW, DIM), jnp.float32),
            pltpu.VMEM((_BPW,), jnp.float32),
            pltpu.VMEM((_BPW, 128), jnp.float32),
            pltpu.VMEM((128,), jnp.int32),
            pltpu.VMEM((128, DIM), jnp.float32),
            pltpu.VMEM((128,), jnp.float32),
            pltpu.VMEM((128, 128), jnp.float32),
            pltpu.SemaphoreType.DMA,
            pltpu.SemaphoreType.DMA,
            pltpu.SemaphoreType.DMA,
            pltpu.SemaphoreType.DMA,
        ],
    )


_SNS = 16
_SCHUNK = VPAD // _SNS
_SVREGS = _SCHUNK // 16


def _mono16(gvec):
    mi = lax.bitcast_convert_type(gvec, jnp.int32)
    neg = lax.bitcast_convert_type(mi >> 31, jnp.uint32)
    return lax.bitcast_convert_type(mi, jnp.uint32) ^ (neg | jnp.uint32(0x80000000))


def _valat(vec, idx_splat):
    i16 = lax.iota(jnp.int32, 16)
    return jnp.sum(jnp.where(i16 == idx_splat, vec, 0))


def _sc_select_body(g_hbm, samp_out,
                    gv, hist_v, tot_v, gall_v, gtot_v, res_v,
                    selu_v, seli_v, tselu_v, tseli_v, rankb_v, trank_v, samp_v,
                    sh_tot, sh_res, sh_selu, sh_seli, sh_rank,
                    sem_a, sem_b):
    wid = lax.axis_index("s")
    base = wid * _SCHUNK
    pltpu.sync_copy(g_hbm.at[pl.ds(base, _SCHUNK)], gv)
    i16 = lax.iota(jnp.int32, 16)
    ones16 = jnp.ones((16,), jnp.int32)
    zeros16 = jnp.zeros((16,), jnp.int32)

    prefix = jnp.zeros((16,), jnp.uint32)
    kk = jnp.full((16,), NUM_NEG, jnp.int32)

    for rnd in range(2):
        sh = 24 - 8 * rnd

        def zro(j, c):
            hist_v[pl.ds(j * 16, 16)] = zeros16
            return c
        lax.fori_loop(0, 256, zro, 0)

        if rnd == 0:
            def scan(j, c):
                u = _mono16(gv[pl.ds(j * 16, 16)])
                b = ((u >> jnp.uint32(sh)) & jnp.uint32(0xFF)).astype(jnp.int32)
                plsc.addupdate_scatter(hist_v, [i16 * 256 + b], ones16)
                return c
        else:
            pref_hi = prefix >> jnp.uint32(sh + 8)

            def scan(j, c):
                u = _mono16(gv[pl.ds(j * 16, 16)])
                b = ((u >> jnp.uint32(sh)) & jnp.uint32(0xFF)).astype(jnp.int32)
                m = (u >> jnp.uint32(sh + 8)) == pref_hi
                plsc.addupdate_scatter(hist_v, [i16 * 256 + b], ones16, mask=m)
                return c
        lax.fori_loop(0, _SVREGS, scan, 0)

        def lred(b, c):
            acc = zeros16
            for l in range(16):
                acc = acc + hist_v[pl.ds(l * 256 + b * 16, 16)]
            tot_v[pl.ds(b * 16, 16)] = acc
            return c
        lax.fori_loop(0, 16, lred, 0)
        pltpu.sync_copy(tot_v, sh_tot.at[wid])
        plsc.subcore_barrier()

        @pl.when(wid == 0)
        def _():
            pltpu.sync_copy(sh_tot, gall_v)

            def gred(b, c):
                acc = zeros16
                for t in range(16):
                    acc = acc + gall_v[t, pl.ds(b * 16, 16)]
                gtot_v[pl.ds(b * 16, 16)] = acc
                return c
            lax.fori_loop(0, 16, gred, 0)

            gs = zeros16
            for j in range(16):
                gs = gs + plsc.load_gather(gtot_v, [i16 * 16 + j])
            cumgs = plsc.cumsum(gs)
            tot_all = jnp.full((16,), cumgs[15], jnp.int32)
            sufg = tot_all - cumgs + gs
            gstar = plsc.all_reduce_population_count(sufg >= kk) - 1
            above_g = tot_all - jnp.full((16,), _valat(cumgs, gstar), jnp.int32)
            sub = gtot_v[pl.ds(gstar[0] * 16, 16)]
            cumsub = plsc.cumsum(sub)
            sub_tot = jnp.full((16,), cumsub[15], jnp.int32)
            sbin = sub_tot - cumsub + sub + above_g
            jstar = plsc.all_reduce_population_count(sbin >= kk) - 1
            byte = gstar * 16 + jstar
            cnt_above = (jnp.full((16,), _valat(sbin, jstar), jnp.int32)
                         - jnp.full((16,), _valat(sub, jstar), jnp.int32))
            k_new = kk - cnt_above
            pref_new = prefix | (byte.astype(jnp.uint32) << jnp.uint32(sh))
            res_v[pl.ds(0, 16)] = lax.bitcast_convert_type(pref_new, jnp.int32)
            res_v[pl.ds(16, 16)] = k_new
            pltpu.sync_copy(res_v, sh_res)

        plsc.subcore_barrier()
        pltpu.sync_copy(sh_res, res_v)
        prefix = lax.bitcast_convert_type(res_v[pl.ds(0, 16)], jnp.uint32)
        kk = res_v[pl.ds(16, 16)]

    p16 = prefix >> jnp.uint32(16)
    selu_v[pl.ds(0, 16)] = jnp.zeros((16,), jnp.uint32)
    selu_v[pl.ds(16, 16)] = jnp.zeros((16,), jnp.uint32)
    seli_v[pl.ds(0, 16)] = zeros16
    seli_v[pl.ds(16, 16)] = zeros16

    def ext(j, off):
        u = _mono16(gv[pl.ds(j * 16, 16)])
        m = (u >> jnp.uint32(16)) >= p16
        plsc.store_compressed(selu_v.at[pl.ds(off, 16)], u, mask=m)
        gidx = jnp.full((16,), base, jnp.int32) + j * 16 + i16
        plsc.store_compressed(seli_v.at[pl.ds(off, 16)], gidx, mask=m)
        cnt = plsc.all_reduce_population_count(m)
        return jnp.minimum(off + cnt[0], 16)
    lax.fori_loop(0, _SVREGS, ext, jnp.int32(0))

    pltpu.sync_copy(selu_v, sh_selu.at[wid])
    pltpu.sync_copy(seli_v, sh_seli.at[wid])
    plsc.subcore_barrier()
    pltpu.sync_copy(sh_selu, tselu_v)
    pltpu.sync_copy(sh_seli, tseli_v)

    def z128(j, c):
        rankb_v[pl.ds(j * 16, 16)] = zeros16
        return c
    lax.fori_loop(0, 8, z128, 0)

    myu = selu_v[pl.ds(0, 16)]
    myidx = seli_v[pl.ds(0, 16)]
    ou = [tselu_v[t, pl.ds(v * 16, 16)] for t in range(16) for v in range(2)]
    oi = [tseli_v[t, pl.ds(v * 16, 16)] for t in range(16) for v in range(2)]
    ranks = zeros16
    for lane in range(16):
        us = jnp.full((16,), myu[lane], jnp.uint32)
        ix = jnp.full((16,), myidx[lane], jnp.int32)
        acc = zeros16
        for q in range(32):
            gt = ou[q] > us
            tie = (ou[q] == us) & (oi[q] < ix)
            acc = acc + gt.astype(jnp.int32) + tie.astype(jnp.int32)
        ranks = jnp.where(i16 == lane, jnp.full((16,), jnp.sum(acc), jnp.int32),
                          ranks)
    plsc.store_scatter(rankb_v, [ranks], myidx, mask=myu != jnp.uint32(0))
    pltpu.sync_copy(rankb_v, sh_rank.at[wid])
    plsc.subcore_barrier()

    @pl.when(wid == 0)
    def _():
        pltpu.sync_copy(sh_rank, trank_v)
        for v in range(8):
            acc = zeros16
            for t in range(16):
                acc = acc + trank_v[t, pl.ds(v * 16, 16)]
            samp_v[pl.ds(v * 16, 16)] = acc
        pltpu.sync_copy(samp_v, samp_out)


@functools.lru_cache(maxsize=1)
def _sc_select_kernel():
    return pl.kernel(
        _sc_select_body,
        mesh=plsc.VectorSubcoreMesh(core_axis_name="c", subcore_axis_name="s",
                                    num_cores=1),
        compiler_params=pltpu.CompilerParams(use_tc_tiling_on_sc=False,
                                             needs_layout_passes=False),
        out_type=jax.ShapeDtypeStruct((128,), jnp.int32),
        scratch_types=[
            pltpu.VMEM((_SCHUNK,), jnp.float32),
            pltpu.VMEM((4096,), jnp.int32),
            pltpu.VMEM((256,), jnp.int32),
            pltpu.VMEM((16, 256), jnp.int32),
            pltpu.VMEM((256,), jnp.int32),
            pltpu.VMEM((32,), jnp.int32),
            pltpu.VMEM((32,), jnp.uint32),
            pltpu.VMEM((32,), jnp.int32),
            pltpu.VMEM((16, 32), jnp.uint32),
            pltpu.VMEM((16, 32), jnp.int32),
            pltpu.VMEM((128,), jnp.int32),
            pltpu.VMEM((16, 128), jnp.int32),
            pltpu.VMEM((128,), jnp.int32),
            pltpu.VMEM_SHARED((16, 256), jnp.int32),
            pltpu.VMEM_SHARED((32,), jnp.int32),
            pltpu.VMEM_SHARED((16, 32), jnp.uint32),
            pltpu.VMEM_SHARED((16, 32), jnp.int32),
            pltpu.VMEM_SHARED((16, 128), jnp.int32),
            pltpu.SemaphoreType.DMA,
            pltpu.SemaphoreType.DMA,
        ],
    )


_BB = 2048
_GRID = B // _BB


def _batch_body(inp_ref, big_ref, sp_ref, out_ref, loss_ref):
    i = pl.program_id(0)
    x = inp_ref[...]
    xa = jnp.concatenate(
        [x, jnp.ones((x.shape[0], 1), jnp.float32)], axis=1)
    twa = big_ref[...][:, :DIM + 1]
    tl = jnp.sum(xa * twa, axis=1, keepdims=True)

    spa = sp_ref[...][:, :DIM + 1]
    sl = lax.dot_general(xa, spa, (((1,), (1,)), ((), ())),
                         preferred_element_type=jnp.float32)

    z = sl - tl
    sp = jnp.maximum(z, 0.0) + jnp.log(1.0 + jnp.exp(-jnp.abs(z)))
    colmask = lax.broadcasted_iota(jnp.int32, sp.shape, 1) < NUM_NEG
    part = jnp.sum(jnp.where(colmask, sp, 0.0)) * jnp.float32(1.0 / (B * NUM_NEG))

    @pl.when(i == 0)
    def _():
        loss_ref[0, 0] = jnp.float32(0.0)

    loss_ref[0, 0] += part
    out_ref[...] = jnp.concatenate([tl, sl[:, :127]], axis=1)


def _batch_call(inp, big, sp):
    return pl.pallas_call(
        _batch_body,
        grid=(_GRID,),
        in_specs=[
            pl.BlockSpec((_BB, DIM), lambda i: (i, 0)),
            pl.BlockSpec((_BB, 128), lambda i: (i, 0)),
            pl.BlockSpec((128, 128), lambda i: (0, 0)),
        ],
        out_specs=[
            pl.BlockSpec((_BB, 128), lambda i: (i, 0)),
            pl.BlockSpec(memory_space=pltpu.SMEM),
        ],
        out_shape=(
            jax.ShapeDtypeStruct((B, 128), jnp.float32),
            jax.ShapeDtypeStruct((1, 1), jnp.float32),
        ),
    )(inp, big, sp)


def kernel(label, inputs, table, biases, counts):
    cext = jnp.concatenate(
        [jnp.ones((1,), jnp.float32), counts,
         jnp.ones((VPAD - VOCAB - 1,), jnp.float32)]).reshape(VROWS, 128)
    bpad = jnp.pad(biases, (0, VPAD - VOCAB - 1)).reshape(VROWS, 128)
    g2d, adj2d = _vocab_call(cext, bpad)

    g_flat = g2d.reshape(-1)
    samp_pad = _sc_select_kernel()(g_flat)

    adj_flat = adj2d.reshape(-1)
    big, sp = _sc_gather_kernel()(table, adj_flat, label, samp_pad)

    logits_pad, loss = _batch_call(inputs, big, sp)
    return logits_pad[:, :NUM_NEG + 1], loss[0, 0]


# speedup vs baseline: 1.9634x; 1.0136x over previous
import functools

import jax
import jax.numpy as jnp
import numpy as np
from jax import lax
from jax.experimental import pallas as pl
from jax.experimental.pallas import tpu as pltpu
from jax.experimental.pallas import tpu_sc as plsc

VOCAB = 100000
DIM = 64
B = 16384
NUM_NEG = 100

VPAD = 100352
VROWS = VPAD // 128

_NC = 2
_NS = 16
_NW = _NC * _NS
_BPW = B // _NW

_TINY = np.float32(np.finfo(np.float32).tiny)


def _threefry_bits(x1):
    ks0 = jnp.uint32(0)
    ks1 = jnp.uint32(42)
    ks2 = jnp.uint32(0x1BD11BDA) ^ ks0 ^ ks1
    rot = ((13, 15, 26, 6), (17, 29, 16, 24))
    ks = (ks0, ks1, ks2)
    x0 = jnp.zeros_like(x1) + ks0
    x1 = x1 + ks1
    for blk in range(5):
        for r in rot[blk % 2]:
            x0 = x0 + x1
            x1 = ((x1 << r) | (x1 >> (32 - r))) ^ x0
        x0 = x0 + ks[(blk + 1) % 3]
        x1 = x1 + ks[(blk + 2) % 3] + jnp.uint32(blk + 1)
    return x0 ^ x1


def _expected(p):
    t = jnp.float32(NUM_NEG) * (-p * (1.0 + p * (0.5 + p * (1.0 / 3.0))))
    return -t * (1.0 + t * (0.5 + t * (1.0 / 6.0)))


def _vocab_body(cext_ref, bias_ref, g_ref, adj_ref):
    c = cext_ref[...]
    rid = lax.broadcasted_iota(jnp.int32, c.shape, 0)
    cid = lax.broadcasted_iota(jnp.int32, c.shape, 1)
    gid = rid * 128 + cid
    valid = (gid >= 1) & (gid <= VOCAB)
    logpz = jnp.float32(0.4) * jnp.log(c)
    pz = jnp.where(valid, jnp.exp(logpz), jnp.float32(0.0))
    s = jnp.sum(pz)
    adj = bias_ref[...] - jnp.log(_expected(pz / s))
    adj_ref[...] = jnp.where(valid, adj, jnp.float32(0.0))
    bits = _threefry_bits(gid.astype(jnp.uint32))
    fb = lax.bitcast_convert_type(
        (bits >> 9) | jnp.uint32(0x3F800000), jnp.float32) - jnp.float32(1.0)
    u = jnp.maximum(fb + _TINY, _TINY)
    gum = -jnp.log(-jnp.log(u))
    g_ref[...] = jnp.where(valid, gum + logpz, jnp.float32(-3e38))


def _vocab_call(cext, bpad):
    return pl.pallas_call(
        _vocab_body,
        in_specs=[
            pl.BlockSpec((VROWS, 128), lambda: (0, 0)),
            pl.BlockSpec((VROWS, 128), lambda: (0, 0)),
        ],
        out_specs=[
            pl.BlockSpec((VROWS, 128), lambda: (0, 0)),
            pl.BlockSpec((VROWS, 128), lambda: (0, 0)),
        ],
        out_shape=(
            jax.ShapeDtypeStruct((VROWS, 128), jnp.float32),
            jax.ShapeDtypeStruct((VROWS, 128), jnp.float32),
        ),
    )(cext, bpad)


def _sc_gather_body(table_hbm, adj_hbm, idx_hbm, samp_hbm, big_out, sp_out,
                    idx_v, rows_v, adj_v, big_v, sidx_v, srows_v, sadj_v, sbig_v,
                    sem_a, sem_b, sem_c, sem_d):
    wid = lax.axis_index("s") * _NC + lax.axis_index("c")
    base = wid * _BPW
    pltpu.sync_copy(idx_hbm.at[pl.ds(base, _BPW)], idx_v)
    cp_a = pltpu.async_copy(table_hbm.at[idx_v], rows_v, sem_a)
    cp_b = pltpu.async_copy(adj_hbm.at[idx_v], adj_v, sem_b)
    cp_a.wait()
    cp_b.wait()

    def move16(j, carry):
        vals = adj_v[pl.ds(j * 16, 16)]
        for k in range(16):
            r = j * 16 + k
            big_v[r, pl.ds(0, 16)] = rows_v[r, pl.ds(0, 16)]
            big_v[r, pl.ds(16, 16)] = rows_v[r, pl.ds(16, 16)]
            big_v[r, pl.ds(32, 16)] = rows_v[r, pl.ds(32, 16)]
            big_v[r, pl.ds(48, 16)] = rows_v[r, pl.ds(48, 16)]
            big_v[r, pl.ds(64, 16)] = jnp.full((16,), vals[k], jnp.float32)
        return carry
    lax.fori_loop(0, _BPW // 16, move16, 0)
    pltpu.sync_copy(big_v, big_out.at[pl.ds(base, _BPW)])

    @pl.when(wid == 0)
    def _():
        pltpu.sync_copy(samp_hbm, sidx_v)
        cp_c = pltpu.async_copy(table_hbm.at[sidx_v], srows_v, sem_c)
        cp_d = pltpu.async_copy(adj_hbm.at[sidx_v], sadj_v, sem_d)
        cp_c.wait()
        cp_d.wait()

        def smove16(j, carry):
            svals = sadj_v[pl.ds(j * 16, 16)]
            for k in range(16):
                r = j * 16 + k
                sbig_v[r, pl.ds(0, 16)] = srows_v[r, pl.ds(0, 16)]
                sbig_v[r, pl.ds(16, 16)] = srows_v[r, pl.ds(16, 16)]
                sbig_v[r, pl.ds(32, 16)] = srows_v[r, pl.ds(32, 16)]
                sbig_v[r, pl.ds(48, 16)] = srows_v[r, pl.ds(48, 16)]
                sbig_v[r, pl.ds(64, 16)] = jnp.full((16,), svals[k], jnp.float32)
            return carry
        lax.fori_loop(0, 8, smove16, 0)
        pltpu.sync_copy(sbig_v, sp_out)


@functools.lru_cache(maxsize=1)
def _sc_gather_kernel():
    return pl.kernel(
        _sc_gather_body,
        mesh=plsc.VectorSubcoreMesh(core_axis_name="c", subcore_axis_name="s"),
        compiler_params=pltpu.CompilerParams(use_tc_tiling_on_sc=False),
        out_type=(
            jax.ShapeDtypeStruct((B, 128), jnp.float32),
            jax.ShapeDtypeStruct((128, 128), jnp.float32),
        ),
        scratch_types=[
            pltpu.VMEM((_BPW,), jnp.int32),
            pltpu.VMEM((_BPW, DIM), jnp.float32),
            pltpu.VMEM((_BPW,), jnp.float32),
            pltpu.VMEM((_BPW, 128), jnp.float32),
            pltpu.VMEM((128,), jnp.int32),
            pltpu.VMEM((128, DIM), jnp.float32),
            pltpu.VMEM((128,), jnp.float32),
            pltpu.VMEM((128, 128), jnp.float32),
            pltpu.SemaphoreType.DMA,
            pltpu.SemaphoreType.DMA,
            pltpu.SemaphoreType.DMA,
            pltpu.SemaphoreType.DMA,
        ],
    )


_SNS = 16
_SCHUNK = VPAD // _SNS
_SVREGS = _SCHUNK // 16


def _mono16(gvec):
    mi = lax.bitcast_convert_type(gvec, jnp.int32)
    neg = lax.bitcast_convert_type(mi >> 31, jnp.uint32)
    return lax.bitcast_convert_type(mi, jnp.uint32) ^ (neg | jnp.uint32(0x80000000))


def _valat(vec, idx_splat):
    i16 = lax.iota(jnp.int32, 16)
    return jnp.sum(jnp.where(i16 == idx_splat, vec, 0))


def _sc_select_body(g_hbm, samp_out,
                    gv, hist_v, tot_v, gall_v, gtot_v, res_v,
                    selu_v, seli_v, tselu_v, tseli_v, rankb_v, trank_v, samp_v,
                    sh_tot, sh_res, sh_selu, sh_seli, sh_rank,
                    sem_a, sem_b):
    wid = lax.axis_index("s")
    base = wid * _SCHUNK
    pltpu.sync_copy(g_hbm.at[pl.ds(base, _SCHUNK)], gv)
    i16 = lax.iota(jnp.int32, 16)
    ones16 = jnp.ones((16,), jnp.int32)
    zeros16 = jnp.zeros((16,), jnp.int32)

    prefix = jnp.zeros((16,), jnp.uint32)
    kk = jnp.full((16,), NUM_NEG, jnp.int32)

    for rnd in range(2):
        sh = 24 - 8 * rnd

        def zro(j, c):
            hist_v[pl.ds(j * 16, 16)] = zeros16
            return c
        lax.fori_loop(0, 256, zro, 0)

        if rnd == 0:
            def scan(j, c):
                u = _mono16(gv[pl.ds(j * 16, 16)])
                b = ((u >> jnp.uint32(sh)) & jnp.uint32(0xFF)).astype(jnp.int32)
                plsc.addupdate_scatter(hist_v, [i16 * 256 + b], ones16)
                return c
        else:
            pref_hi = prefix >> jnp.uint32(sh + 8)

            def scan(j, c):
                u = _mono16(gv[pl.ds(j * 16, 16)])
                b = ((u >> jnp.uint32(sh)) & jnp.uint32(0xFF)).astype(jnp.int32)
                m = (u >> jnp.uint32(sh + 8)) == pref_hi
                plsc.addupdate_scatter(hist_v, [i16 * 256 + b], ones16, mask=m)
                return c
        lax.fori_loop(0, _SVREGS, scan, 0)

        def lred(b, c):
            acc = zeros16
            for l in range(16):
                acc = acc + hist_v[pl.ds(l * 256 + b * 16, 16)]
            tot_v[pl.ds(b * 16, 16)] = acc
            return c
        lax.fori_loop(0, 16, lred, 0)
        pltpu.sync_copy(tot_v, sh_tot.at[wid])
        plsc.subcore_barrier()

        @pl.when(wid == 0)
        def _():
            pltpu.sync_copy(sh_tot, gall_v)

            def gred(b, c):
                acc = zeros16
                for t in range(16):
                    acc = acc + gall_v[t, pl.ds(b * 16, 16)]
                gtot_v[pl.ds(b * 16, 16)] = acc
                return c
            lax.fori_loop(0, 16, gred, 0)

            gs = zeros16
            for j in range(16):
                gs = gs + plsc.load_gather(gtot_v, [i16 * 16 + j])
            cumgs = plsc.cumsum(gs)
            tot_all = jnp.full((16,), cumgs[15], jnp.int32)
            sufg = tot_all - cumgs + gs
            gstar = plsc.all_reduce_population_count(sufg >= kk) - 1
            above_g = tot_all - jnp.full((16,), _valat(cumgs, gstar), jnp.int32)
            sub = gtot_v[pl.ds(gstar[0] * 16, 16)]
            cumsub = plsc.cumsum(sub)
            sub_tot = jnp.full((16,), cumsub[15], jnp.int32)
            sbin = sub_tot - cumsub + sub + above_g
            jstar = plsc.all_reduce_population_count(sbin >= kk) - 1
            byte = gstar * 16 + jstar
            cnt_above = (jnp.full((16,), _valat(sbin, jstar), jnp.int32)
                         - jnp.full((16,), _valat(sub, jstar), jnp.int32))
            k_new = kk - cnt_above
            pref_new = prefix | (byte.astype(jnp.uint32) << jnp.uint32(sh))
            res_v[pl.ds(0, 16)] = lax.bitcast_convert_type(pref_new, jnp.int32)
            res_v[pl.ds(16, 16)] = k_new
            pltpu.sync_copy(res_v, sh_res)

        plsc.subcore_barrier()
        pltpu.sync_copy(sh_res, res_v)
        prefix = lax.bitcast_convert_type(res_v[pl.ds(0, 16)], jnp.uint32)
        kk = res_v[pl.ds(16, 16)]

    p16 = prefix >> jnp.uint32(16)
    selu_v[pl.ds(0, 16)] = jnp.zeros((16,), jnp.uint32)
    selu_v[pl.ds(16, 16)] = jnp.zeros((16,), jnp.uint32)
    seli_v[pl.ds(0, 16)] = zeros16
    seli_v[pl.ds(16, 16)] = zeros16

    def ext(j, off):
        u = _mono16(gv[pl.ds(j * 16, 16)])
        m = (u >> jnp.uint32(16)) >= p16
        plsc.store_compressed(selu_v.at[pl.ds(off, 16)], u, mask=m)
        gidx = jnp.full((16,), base, jnp.int32) + j * 16 + i16
        plsc.store_compressed(seli_v.at[pl.ds(off, 16)], gidx, mask=m)
        cnt = plsc.all_reduce_population_count(m)
        return jnp.minimum(off + cnt[0], 16)
    lax.fori_loop(0, _SVREGS, ext, jnp.int32(0))

    pltpu.sync_copy(selu_v, sh_selu.at[wid])
    pltpu.sync_copy(seli_v, sh_seli.at[wid])
    plsc.subcore_barrier()
    pltpu.sync_copy(sh_selu, tselu_v)
    pltpu.sync_copy(sh_seli, tseli_v)

    def z128(j, c):
        rankb_v[pl.ds(j * 16, 16)] = zeros16
        return c
    lax.fori_loop(0, 8, z128, 0)

    myu = selu_v[pl.ds(0, 16)]
    myidx = seli_v[pl.ds(0, 16)]
    ou = [tselu_v[t, pl.ds(v * 16, 16)] for t in range(16) for v in range(2)]
    oi = [tseli_v[t, pl.ds(v * 16, 16)] for t in range(16) for v in range(2)]
    ranks = zeros16
    for lane in range(16):
        us = jnp.full((16,), myu[lane], jnp.uint32)
        ix = jnp.full((16,), myidx[lane], jnp.int32)
        acc = zeros16
        for q in range(32):
            gt = ou[q] > us
            tie = (ou[q] == us) & (oi[q] < ix)
            acc = acc + gt.astype(jnp.int32) + tie.astype(jnp.int32)
        ranks = jnp.where(i16 == lane, jnp.full((16,), jnp.sum(acc), jnp.int32),
                          ranks)
    plsc.store_scatter(rankb_v, [ranks], myidx, mask=myu != jnp.uint32(0))
    pltpu.sync_copy(rankb_v, sh_rank.at[wid])
    plsc.subcore_barrier()

    @pl.when(wid == 0)
    def _():
        pltpu.sync_copy(sh_rank, trank_v)
        for v in range(8):
            acc = zeros16
            for t in range(16):
                acc = acc + trank_v[t, pl.ds(v * 16, 16)]
            samp_v[pl.ds(v * 16, 16)] = acc
        pltpu.sync_copy(samp_v, samp_out)


@functools.lru_cache(maxsize=1)
def _sc_select_kernel():
    return pl.kernel(
        _sc_select_body,
        mesh=plsc.VectorSubcoreMesh(core_axis_name="c", subcore_axis_name="s",
                                    num_cores=1),
        compiler_params=pltpu.CompilerParams(use_tc_tiling_on_sc=False,
                                             needs_layout_passes=False),
        out_type=jax.ShapeDtypeStruct((128,), jnp.int32),
        scratch_types=[
            pltpu.VMEM((_SCHUNK,), jnp.float32),
            pltpu.VMEM((4096,), jnp.int32),
            pltpu.VMEM((256,), jnp.int32),
            pltpu.VMEM((16, 256), jnp.int32),
            pltpu.VMEM((256,), jnp.int32),
            pltpu.VMEM((32,), jnp.int32),
            pltpu.VMEM((32,), jnp.uint32),
            pltpu.VMEM((32,), jnp.int32),
            pltpu.VMEM((16, 32), jnp.uint32),
            pltpu.VMEM((16, 32), jnp.int32),
            pltpu.VMEM((128,), jnp.int32),
            pltpu.VMEM((16, 128), jnp.int32),
            pltpu.VMEM((128,), jnp.int32),
            pltpu.VMEM_SHARED((16, 256), jnp.int32),
            pltpu.VMEM_SHARED((32,), jnp.int32),
            pltpu.VMEM_SHARED((16, 32), jnp.uint32),
            pltpu.VMEM_SHARED((16, 32), jnp.int32),
            pltpu.VMEM_SHARED((16, 128), jnp.int32),
            pltpu.SemaphoreType.DMA,
            pltpu.SemaphoreType.DMA,
        ],
    )


_BB = 4096
_GRID = B // _BB


def _batch_body(inp_ref, big_ref, sp_ref, out_ref, loss_ref):
    i = pl.program_id(0)
    x = inp_ref[...]
    xa = jnp.concatenate(
        [x, jnp.ones((x.shape[0], 1), jnp.float32)], axis=1)
    twa = big_ref[...][:, :DIM + 1]
    tl = jnp.sum(xa * twa, axis=1, keepdims=True)

    spa = sp_ref[...][:, :DIM + 1]
    sl = lax.dot_general(xa, spa, (((1,), (1,)), ((), ())),
                         preferred_element_type=jnp.float32)

    z = sl - tl
    sp = jnp.maximum(z, 0.0) + jnp.log(1.0 + jnp.exp(-jnp.abs(z)))
    colmask = lax.broadcasted_iota(jnp.int32, sp.shape, 1) < NUM_NEG
    part = jnp.sum(jnp.where(colmask, sp, 0.0)) * jnp.float32(1.0 / (B * NUM_NEG))

    @pl.when(i == 0)
    def _():
        loss_ref[0, 0] = jnp.float32(0.0)

    loss_ref[0, 0] += part
    out_ref[...] = jnp.concatenate([tl, sl[:, :127]], axis=1)


def _batch_call(inp, big, sp):
    return pl.pallas_call(
        _batch_body,
        grid=(_GRID,),
        in_specs=[
            pl.BlockSpec((_BB, DIM), lambda i: (i, 0)),
            pl.BlockSpec((_BB, 128), lambda i: (i, 0)),
            pl.BlockSpec((128, 128), lambda i: (0, 0)),
        ],
        out_specs=[
            pl.BlockSpec((_BB, 128), lambda i: (i, 0)),
            pl.BlockSpec(memory_space=pltpu.SMEM),
        ],
        out_shape=(
            jax.ShapeDtypeStruct((B, 128), jnp.float32),
            jax.ShapeDtypeStruct((1, 1), jnp.float32),
        ),
    )(inp, big, sp)


def kernel(label, inputs, table, biases, counts):
    cext = jnp.concatenate(
        [jnp.ones((1,), jnp.float32), counts,
         jnp.ones((VPAD - VOCAB - 1,), jnp.float32)]).reshape(VROWS, 128)
    bpad = jnp.pad(biases, (0, VPAD - VOCAB - 1)).reshape(VROWS, 128)
    g2d, adj2d = _vocab_call(cext, bpad)

    g_flat = g2d.reshape(-1)
    samp_pad = _sc_select_kernel()(g_flat)

    adj_flat = adj2d.reshape(-1)
    big, sp = _sc_gather_kernel()(table, adj_flat, label, samp_pad)

    logits_pad, loss = _batch_call(inputs, big, sp)
    return logits_pad[:, :NUM_NEG + 1], loss[0, 0]


# speedup vs baseline: 1.9697x; 1.0032x over previous
import functools

import jax
import jax.numpy as jnp
import numpy as np
from jax import lax
from jax.experimental import pallas as pl
from jax.experimental.pallas import tpu as pltpu
from jax.experimental.pallas import tpu_sc as plsc

VOCAB = 100000
DIM = 64
B = 16384
NUM_NEG = 100

VPAD = 100352
VROWS = VPAD // 128

_NC = 2
_NS = 16
_NW = _NC * _NS
_BPW = B // _NW

_TINY = np.float32(np.finfo(np.float32).tiny)


def _threefry_bits(x1):
    ks0 = jnp.uint32(0)
    ks1 = jnp.uint32(42)
    ks2 = jnp.uint32(0x1BD11BDA) ^ ks0 ^ ks1
    rot = ((13, 15, 26, 6), (17, 29, 16, 24))
    ks = (ks0, ks1, ks2)
    x0 = jnp.zeros_like(x1) + ks0
    x1 = x1 + ks1
    for blk in range(5):
        for r in rot[blk % 2]:
            x0 = x0 + x1
            x1 = ((x1 << r) | (x1 >> (32 - r))) ^ x0
        x0 = x0 + ks[(blk + 1) % 3]
        x1 = x1 + ks[(blk + 2) % 3] + jnp.uint32(blk + 1)
    return x0 ^ x1


def _expected(p):
    t = jnp.float32(NUM_NEG) * (-p * (1.0 + p * (0.5 + p * (1.0 / 3.0))))
    return -t * (1.0 + t * (0.5 + t * (1.0 / 6.0)))


def _vocab_body(cext_ref, bias_ref, g_ref, adj_ref):
    c = cext_ref[...]
    rid = lax.broadcasted_iota(jnp.int32, c.shape, 0)
    cid = lax.broadcasted_iota(jnp.int32, c.shape, 1)
    gid = rid * 128 + cid
    valid = (gid >= 1) & (gid <= VOCAB)
    logpz = jnp.float32(0.4) * jnp.log(c)
    pz = jnp.where(valid, jnp.exp(logpz), jnp.float32(0.0))
    s = jnp.sum(pz)
    adj = bias_ref[...] - jnp.log(_expected(pz / s))
    adj_ref[...] = jnp.where(valid, adj, jnp.float32(0.0))
    bits = _threefry_bits(gid.astype(jnp.uint32))
    fb = lax.bitcast_convert_type(
        (bits >> 9) | jnp.uint32(0x3F800000), jnp.float32) - jnp.float32(1.0)
    u = jnp.maximum(fb + _TINY, _TINY)
    gum = -jnp.log(-jnp.log(u))
    g_ref[...] = jnp.where(valid, gum + logpz, jnp.float32(-3e38))


def _vocab_call(cext, bpad):
    return pl.pallas_call(
        _vocab_body,
        in_specs=[
            pl.BlockSpec((VROWS, 128), lambda: (0, 0)),
            pl.BlockSpec((VROWS, 128), lambda: (0, 0)),
        ],
        out_specs=[
            pl.BlockSpec((VROWS, 128), lambda: (0, 0)),
            pl.BlockSpec((VROWS, 128), lambda: (0, 0)),
        ],
        out_shape=(
            jax.ShapeDtypeStruct((VROWS, 128), jnp.float32),
            jax.ShapeDtypeStruct((VROWS, 128), jnp.float32),
        ),
    )(cext, bpad)


def _sc_gather_body(table_hbm, adj_hbm, idx_hbm, samp_hbm, big_out, sp_out,
                    idx_v, rows_v, adj_v, big_v, sidx_v, srows_v, sadj_v, sbig_v,
                    sem_a, sem_b, sem_c, sem_d):
    wid = lax.axis_index("s") * _NC + lax.axis_index("c")
    base = wid * _BPW
    pltpu.sync_copy(idx_hbm.at[pl.ds(base, _BPW)], idx_v)
    cp_a = pltpu.async_copy(table_hbm.at[idx_v], rows_v, sem_a)
    cp_b = pltpu.async_copy(adj_hbm.at[idx_v], adj_v, sem_b)
    cp_a.wait()
    cp_b.wait()

    def move16(j, carry):
        vals = adj_v[pl.ds(j * 16, 16)]
        for k in range(16):
            r = j * 16 + k
            big_v[r, pl.ds(0, 16)] = rows_v[r, pl.ds(0, 16)]
            big_v[r, pl.ds(16, 16)] = rows_v[r, pl.ds(16, 16)]
            big_v[r, pl.ds(32, 16)] = rows_v[r, pl.ds(32, 16)]
            big_v[r, pl.ds(48, 16)] = rows_v[r, pl.ds(48, 16)]
            big_v[r, pl.ds(64, 16)] = jnp.full((16,), vals[k], jnp.float32)
        return carry
    lax.fori_loop(0, _BPW // 16, move16, 0)
    pltpu.sync_copy(big_v, big_out.at[pl.ds(base, _BPW)])

    @pl.when(wid == 0)
    def _():
        pltpu.sync_copy(samp_hbm, sidx_v)
        cp_c = pltpu.async_copy(table_hbm.at[sidx_v], srows_v, sem_c)
        cp_d = pltpu.async_copy(adj_hbm.at[sidx_v], sadj_v, sem_d)
        cp_c.wait()
        cp_d.wait()

        def smove16(j, carry):
            svals = sadj_v[pl.ds(j * 16, 16)]
            for k in range(16):
                r = j * 16 + k
                sbig_v[r, pl.ds(0, 16)] = srows_v[r, pl.ds(0, 16)]
                sbig_v[r, pl.ds(16, 16)] = srows_v[r, pl.ds(16, 16)]
                sbig_v[r, pl.ds(32, 16)] = srows_v[r, pl.ds(32, 16)]
                sbig_v[r, pl.ds(48, 16)] = srows_v[r, pl.ds(48, 16)]
                sbig_v[r, pl.ds(64, 16)] = jnp.full((16,), svals[k], jnp.float32)
            return carry
        lax.fori_loop(0, 8, smove16, 0)
        pltpu.sync_copy(sbig_v, sp_out)


@functools.lru_cache(maxsize=1)
def _sc_gather_kernel():
    return pl.kernel(
        _sc_gather_body,
        mesh=plsc.VectorSubcoreMesh(core_axis_name="c", subcore_axis_name="s"),
        compiler_params=pltpu.CompilerParams(use_tc_tiling_on_sc=False),
        out_type=(
            jax.ShapeDtypeStruct((B, 128), jnp.float32),
            jax.ShapeDtypeStruct((128, 128), jnp.float32),
        ),
        scratch_types=[
            pltpu.VMEM((_BPW,), jnp.int32),
            pltpu.VMEM((_BPW, DIM), jnp.float32),
            pltpu.VMEM((_BPW,), jnp.float32),
            pltpu.VMEM((_BPW, 128), jnp.float32),
            pltpu.VMEM((128,), jnp.int32),
            pltpu.VMEM((128, DIM), jnp.float32),
            pltpu.VMEM((128,), jnp.float32),
            pltpu.VMEM((128, 128), jnp.float32),
            pltpu.SemaphoreType.DMA,
            pltpu.SemaphoreType.DMA,
            pltpu.SemaphoreType.DMA,
            pltpu.SemaphoreType.DMA,
        ],
    )


_SNS = 16
_SCHUNK = VPAD // _SNS
_SVREGS = _SCHUNK // 16


def _mono16(gvec):
    mi = lax.bitcast_convert_type(gvec, jnp.int32)
    neg = lax.bitcast_convert_type(mi >> 31, jnp.uint32)
    return lax.bitcast_convert_type(mi, jnp.uint32) ^ (neg | jnp.uint32(0x80000000))


def _valat(vec, idx_splat):
    i16 = lax.iota(jnp.int32, 16)
    return jnp.sum(jnp.where(i16 == idx_splat, vec, 0))


def _sc_select_body(g_hbm, samp_out,
                    gv, hist_v, tot_v, gall_v, gtot_v, res_v,
                    selu_v, seli_v, tselu_v, tseli_v, rankb_v, trank_v, samp_v,
                    sh_tot, sh_res, sh_selu, sh_seli, sh_rank,
                    sem_a, sem_b):
    wid = lax.axis_index("s")
    base = wid * _SCHUNK
    pltpu.sync_copy(g_hbm.at[pl.ds(wid * (VROWS // _SNS), VROWS // _SNS)], gv)
    i16 = lax.iota(jnp.int32, 16)
    ones16 = jnp.ones((16,), jnp.int32)
    zeros16 = jnp.zeros((16,), jnp.int32)

    prefix = jnp.zeros((16,), jnp.uint32)
    kk = jnp.full((16,), NUM_NEG, jnp.int32)

    for rnd in range(2):
        sh = 24 - 8 * rnd

        def zro(j, c):
            hist_v[pl.ds(j * 16, 16)] = zeros16
            return c
        lax.fori_loop(0, 256, zro, 0)

        if rnd == 0:
            def scan(j, c):
                u = _mono16(gv[j // 8, pl.ds((j % 8) * 16, 16)])
                b = ((u >> jnp.uint32(sh)) & jnp.uint32(0xFF)).astype(jnp.int32)
                plsc.addupdate_scatter(hist_v, [i16 * 256 + b], ones16)
                return c
        else:
            pref_hi = prefix >> jnp.uint32(sh + 8)

            def scan(j, c):
                u = _mono16(gv[j // 8, pl.ds((j % 8) * 16, 16)])
                b = ((u >> jnp.uint32(sh)) & jnp.uint32(0xFF)).astype(jnp.int32)
                m = (u >> jnp.uint32(sh + 8)) == pref_hi
                plsc.addupdate_scatter(hist_v, [i16 * 256 + b], ones16, mask=m)
                return c
        lax.fori_loop(0, _SVREGS, scan, 0)

        def lred(b, c):
            acc = zeros16
            for l in range(16):
                acc = acc + hist_v[pl.ds(l * 256 + b * 16, 16)]
            tot_v[pl.ds(b * 16, 16)] = acc
            return c
        lax.fori_loop(0, 16, lred, 0)
        pltpu.sync_copy(tot_v, sh_tot.at[wid])
        plsc.subcore_barrier()

        @pl.when(wid == 0)
        def _():
            pltpu.sync_copy(sh_tot, gall_v)

            def gred(b, c):
                acc = zeros16
                for t in range(16):
                    acc = acc + gall_v[t, pl.ds(b * 16, 16)]
                gtot_v[pl.ds(b * 16, 16)] = acc
                return c
            lax.fori_loop(0, 16, gred, 0)

            gs = zeros16
            for j in range(16):
                gs = gs + plsc.load_gather(gtot_v, [i16 * 16 + j])
            cumgs = plsc.cumsum(gs)
            tot_all = jnp.full((16,), cumgs[15], jnp.int32)
            sufg = tot_all - cumgs + gs
            gstar = plsc.all_reduce_population_count(sufg >= kk) - 1
            above_g = tot_all - jnp.full((16,), _valat(cumgs, gstar), jnp.int32)
            sub = gtot_v[pl.ds(gstar[0] * 16, 16)]
            cumsub = plsc.cumsum(sub)
            sub_tot = jnp.full((16,), cumsub[15], jnp.int32)
            sbin = sub_tot - cumsub + sub + above_g
            jstar = plsc.all_reduce_population_count(sbin >= kk) - 1
            byte = gstar * 16 + jstar
            cnt_above = (jnp.full((16,), _valat(sbin, jstar), jnp.int32)
                         - jnp.full((16,), _valat(sub, jstar), jnp.int32))
            k_new = kk - cnt_above
            pref_new = prefix | (byte.astype(jnp.uint32) << jnp.uint32(sh))
            res_v[pl.ds(0, 16)] = lax.bitcast_convert_type(pref_new, jnp.int32)
            res_v[pl.ds(16, 16)] = k_new
            pltpu.sync_copy(res_v, sh_res)

        plsc.subcore_barrier()
        pltpu.sync_copy(sh_res, res_v)
        prefix = lax.bitcast_convert_type(res_v[pl.ds(0, 16)], jnp.uint32)
        kk = res_v[pl.ds(16, 16)]

    p16 = prefix >> jnp.uint32(16)
    selu_v[pl.ds(0, 16)] = jnp.zeros((16,), jnp.uint32)
    selu_v[pl.ds(16, 16)] = jnp.zeros((16,), jnp.uint32)
    seli_v[pl.ds(0, 16)] = zeros16
    seli_v[pl.ds(16, 16)] = zeros16

    def ext(j, off):
        u = _mono16(gv[j // 8, pl.ds((j % 8) * 16, 16)])
        m = (u >> jnp.uint32(16)) >= p16
        plsc.store_compressed(selu_v.at[pl.ds(off, 16)], u, mask=m)
        gidx = jnp.full((16,), base, jnp.int32) + j * 16 + i16
        plsc.store_compressed(seli_v.at[pl.ds(off, 16)], gidx, mask=m)
        cnt = plsc.all_reduce_population_count(m)
        return jnp.minimum(off + cnt[0], 16)
    lax.fori_loop(0, _SVREGS, ext, jnp.int32(0))

    pltpu.sync_copy(selu_v, sh_selu.at[wid])
    pltpu.sync_copy(seli_v, sh_seli.at[wid])
    plsc.subcore_barrier()
    pltpu.sync_copy(sh_selu, tselu_v)
    pltpu.sync_copy(sh_seli, tseli_v)

    def z128(j, c):
        rankb_v[pl.ds(j * 16, 16)] = zeros16
        return c
    lax.fori_loop(0, 8, z128, 0)

    myu = selu_v[pl.ds(0, 16)]
    myidx = seli_v[pl.ds(0, 16)]
    ou = [tselu_v[t, pl.ds(v * 16, 16)] for t in range(16) for v in range(2)]
    oi = [tseli_v[t, pl.ds(v * 16, 16)] for t in range(16) for v in range(2)]
    ranks = zeros16
    for lane in range(16):
        us = jnp.full((16,), myu[lane], jnp.uint32)
        ix = jnp.full((16,), myidx[lane], jnp.int32)
        acc = zeros16
        for q in range(32):
            gt = ou[q] > us
            tie = (ou[q] == us) & (oi[q] < ix)
            acc = acc + gt.astype(jnp.int32) + tie.astype(jnp.int32)
        ranks = jnp.where(i16 == lane, jnp.full((16,), jnp.sum(acc), jnp.int32),
                          ranks)
    plsc.store_scatter(rankb_v, [ranks], myidx, mask=myu != jnp.uint32(0))
    pltpu.sync_copy(rankb_v, sh_rank.at[wid])
    plsc.subcore_barrier()

    @pl.when(wid == 0)
    def _():
        pltpu.sync_copy(sh_rank, trank_v)
        for v in range(8):
            acc = zeros16
            for t in range(16):
                acc = acc + trank_v[t, pl.ds(v * 16, 16)]
            samp_v[pl.ds(v * 16, 16)] = acc
        pltpu.sync_copy(samp_v, samp_out)


@functools.lru_cache(maxsize=1)
def _sc_select_kernel():
    return pl.kernel(
        _sc_select_body,
        mesh=plsc.VectorSubcoreMesh(core_axis_name="c", subcore_axis_name="s",
                                    num_cores=1),
        compiler_params=pltpu.CompilerParams(use_tc_tiling_on_sc=False,
                                             needs_layout_passes=False),
        out_type=jax.ShapeDtypeStruct((128,), jnp.int32),
        scratch_types=[
            pltpu.VMEM((VROWS // _SNS, 128), jnp.float32),
            pltpu.VMEM((4096,), jnp.int32),
            pltpu.VMEM((256,), jnp.int32),
            pltpu.VMEM((16, 256), jnp.int32),
            pltpu.VMEM((256,), jnp.int32),
            pltpu.VMEM((32,), jnp.int32),
            pltpu.VMEM((32,), jnp.uint32),
            pltpu.VMEM((32,), jnp.int32),
            pltpu.VMEM((16, 32), jnp.uint32),
            pltpu.VMEM((16, 32), jnp.int32),
            pltpu.VMEM((128,), jnp.int32),
            pltpu.VMEM((16, 128), jnp.int32),
            pltpu.VMEM((128,), jnp.int32),
            pltpu.VMEM_SHARED((16, 256), jnp.int32),
            pltpu.VMEM_SHARED((32,), jnp.int32),
            pltpu.VMEM_SHARED((16, 32), jnp.uint32),
            pltpu.VMEM_SHARED((16, 32), jnp.int32),
            pltpu.VMEM_SHARED((16, 128), jnp.int32),
            pltpu.SemaphoreType.DMA,
            pltpu.SemaphoreType.DMA,
        ],
    )


_BB = 4096
_GRID = B // _BB


def _batch_body(inp_ref, big_ref, sp_ref, out_ref, loss_ref):
    i = pl.program_id(0)
    x = inp_ref[...]
    xa = jnp.concatenate(
        [x, jnp.ones((x.shape[0], 1), jnp.float32)], axis=1)
    twa = big_ref[...][:, :DIM + 1]
    tl = jnp.sum(xa * twa, axis=1, keepdims=True)

    spa = sp_ref[...][:, :DIM + 1]
    sl = lax.dot_general(xa, spa, (((1,), (1,)), ((), ())),
                         preferred_element_type=jnp.float32)

    z = sl - tl
    sp = jnp.maximum(z, 0.0) + jnp.log(1.0 + jnp.exp(-jnp.abs(z)))
    colmask = lax.broadcasted_iota(jnp.int32, sp.shape, 1) < NUM_NEG
    part = jnp.sum(jnp.where(colmask, sp, 0.0)) * jnp.float32(1.0 / (B * NUM_NEG))

    @pl.when(i == 0)
    def _():
        loss_ref[0, 0] = jnp.float32(0.0)

    loss_ref[0, 0] += part
    out_ref[...] = jnp.concatenate([tl, sl[:, :127]], axis=1)


def _batch_call(inp, big, sp):
    return pl.pallas_call(
        _batch_body,
        grid=(_GRID,),
        in_specs=[
            pl.BlockSpec((_BB, DIM), lambda i: (i, 0)),
            pl.BlockSpec((_BB, 128), lambda i: (i, 0)),
            pl.BlockSpec((128, 128), lambda i: (0, 0)),
        ],
        out_specs=[
            pl.BlockSpec((_BB, 128), lambda i: (i, 0)),
            pl.BlockSpec(memory_space=pltpu.SMEM),
        ],
        out_shape=(
            jax.ShapeDtypeStruct((B, 128), jnp.float32),
            jax.ShapeDtypeStruct((1, 1), jnp.float32),
        ),
    )(inp, big, sp)


def kernel(label, inputs, table, biases, counts):
    cext = jnp.concatenate(
        [jnp.ones((1,), jnp.float32), counts,
         jnp.ones((VPAD - VOCAB - 1,), jnp.float32)]).reshape(VROWS, 128)
    bpad = jnp.pad(biases, (0, VPAD - VOCAB - 1)).reshape(VROWS, 128)
    g2d, adj2d = _vocab_call(cext, bpad)

    samp_pad = _sc_select_kernel()(g2d)

    adj_flat = adj2d.reshape(-1)
    big, sp = _sc_gather_kernel()(table, adj_flat, label, samp_pad)

    logits_pad, loss = _batch_call(inputs, big, sp)
    return logits_pad[:, :NUM_NEG + 1], loss[0, 0]
